# baseline (speedup 1.0000x reference)
import functools

import jax
import jax.numpy as jnp
from jax import lax
from jax.experimental import pallas as pl
from jax.experimental.pallas import tpu as pltpu
from jax.experimental.pallas import tpu_sc as plsc

F32 = jnp.float32
I32 = jnp.int32
_HI = lax.Precision.HIGHEST

NC, NS, LANES = 2, 16, 16
NW = NC * NS

NNODE = 10000
DH = 128
RPT = 632
NPAD = RPT * NS
NB = 8
BR = NPAD // NB
CHUNK = 128
NCHUNK = 82
KE = CHUNK * NCHUNK
EPAD = KE * NW


def _dot(a, b):
    return jnp.dot(a, b, precision=_HI, preferred_element_type=F32)


def _make_edge_kernel():
    nb_full, rem = divmod(RPT, CHUNK)
    mesh = plsc.VectorSubcoreMesh(
        core_axis_name="c", subcore_axis_name="s",
        num_cores=NC, num_subcores=NS)

    @functools.partial(
        pl.kernel,
        out_type=(jax.ShapeDtypeStruct((NC * NPAD, DH), F32),
                  jax.ShapeDtypeStruct((NW * NPAD,), F32)),
        mesh=mesh,
        scratch_types=[
            pltpu.VMEM_SHARED((NPAD, DH), F32),
            pltpu.VMEM((CHUNK,), I32),
            pltpu.VMEM((CHUNK,), I32),
            pltpu.VMEM((CHUNK,), F32),
            pltpu.VMEM((CHUNK, DH), F32),
            pltpu.VMEM((NPAD,), F32),
            pltpu.VMEM((NPAD,), F32),
            pltpu.VMEM((NPAD,), F32),
            pltpu.SemaphoreType.DMA,
        ],
        compiler_params=pltpu.CompilerParams(needs_layout_passes=False),
    )
    def edge_kernel(src_hbm, dst_hbm, ss_hbm, sd_hbm, h_hbm, zrows_hbm, z1_hbm,
                    acc_hbm, den_hbm,
                    acc_sp, sidx, didx, exv, rows, ssv, sdv, denv, sem):
        cid = lax.axis_index("c")
        sid = lax.axis_index("s")
        wid = cid * NS + sid
        r0 = sid * RPT

        pltpu.sync_copy(ss_hbm, ssv)
        pltpu.sync_copy(sd_hbm, sdv)
        pltpu.sync_copy(z1_hbm, denv)

        pltpu.sync_copy(zrows_hbm, rows)
        for i in range(nb_full):
            pltpu.sync_copy(rows, acc_sp.at[pl.ds(r0 + i * CHUNK, CHUNK)])
        pltpu.sync_copy(rows.at[pl.ds(0, rem)],
                        acc_sp.at[pl.ds(r0 + nb_full * CHUNK, rem)])

        plsc.subcore_barrier()

        ebase = wid * KE

        def chunk_body(c, carry):
            base = ebase + c * CHUNK
            pltpu.sync_copy(src_hbm.at[pl.ds(base, CHUNK)], sidx)
            pltpu.sync_copy(dst_hbm.at[pl.ds(base, CHUNK)], didx)
            gcp = pltpu.async_copy(h_hbm.at[sidx], rows, sem)
            for j in range(CHUNK // LANES):
                si = sidx[pl.ds(j * LANES, LANES)]
                di = didx[pl.ds(j * LANES, LANES)]
                t = plsc.load_gather(ssv, [si]) + plsc.load_gather(sdv, [di])
                e = jnp.maximum(t, 0.2 * t)
                ex = jnp.exp(e)
                exv[pl.ds(j * LANES, LANES)] = ex
                plsc.addupdate_scatter(denv, [di], ex)
            gcp.wait()

            def scale_body(jj, c2):
                b = plsc.load_gather(exv, [jnp.full((LANES,), jj, I32)])
                for dd in range(DH // LANES):
                    sl = pl.ds(dd * LANES, LANES)
                    rows[jj, sl] = rows[jj, sl] * b
                return c2
            lax.fori_loop(0, CHUNK, scale_body, 0)

            return carry

        lax.fori_loop(0, NCHUNK, chunk_body, 0)

        pltpu.sync_copy(denv, den_hbm.at[pl.ds(wid * NPAD, NPAD)])

        plsc.subcore_barrier()

        ob = cid * NPAD + r0
        for i in range(nb_full):
            pltpu.sync_copy(acc_sp.at[pl.ds(r0 + i * CHUNK, CHUNK)], rows)
            pltpu.sync_copy(rows, acc_hbm.at[pl.ds(ob + i * CHUNK, CHUNK)])
        pltpu.sync_copy(acc_sp.at[pl.ds(r0 + nb_full * CHUNK, rem)],
                        rows.at[pl.ds(0, rem)])
        pltpu.sync_copy(rows.at[pl.ds(0, rem)],
                        acc_hbm.at[pl.ds(ob + nb_full * CHUNK, rem)])

    return edge_kernel


_EK = _make_edge_kernel()


def _edge_pass(src, dst, ss, sd, h, zrows, z1):
    acc, den = _EK(src, dst, ss.reshape(-1), sd.reshape(-1), h, zrows, z1)
    return acc.reshape(NC, NPAD, DH), den.reshape(NW, NPAD).T


def _t0_body(x_ref, w_ref, as_ref, ad_ref, h_ref, ss_ref, sd_ref):
    x = x_ref[...]
    nrm = jnp.sqrt(jnp.sum(x * x, axis=1, keepdims=True))
    xn = x / jnp.maximum(nrm, 1e-12)
    h = _dot(xn, w_ref[...])
    h_ref[...] = h
    ss_ref[...] = _dot(h, as_ref[...])
    sd_ref[...] = _dot(h, ad_ref[...])


def _row_spec(d):
    return pl.BlockSpec((BR, d), lambda i: (i, 0))


def _full_spec(shape):
    return pl.BlockSpec(shape, lambda i: tuple(0 for _ in shape))


def _t0(x_pad, w, a_s, a_d):
    dh = w.shape[1]
    return pl.pallas_call(
        _t0_body,
        grid=(NB,),
        in_specs=[_row_spec(x_pad.shape[1]), _full_spec(w.shape),
                  _full_spec((w.shape[1], 1)), _full_spec((w.shape[1], 1))],
        out_specs=(_row_spec(dh), _row_spec(1), _row_spec(1)),
        out_shape=(jax.ShapeDtypeStruct((NPAD, dh), F32),
                   jax.ShapeDtypeStruct((NPAD, 1), F32),
                   jax.ShapeDtypeStruct((NPAD, 1), F32)),
    )(x_pad, w, a_s.reshape(-1, 1), a_d.reshape(-1, 1))


def _combine(acc_ref, den_ref, b_ref, d):
    dsum = jnp.sum(den_ref[...], axis=1, keepdims=True) + 1e-16
    agg = (acc_ref[0, :, :d] + acc_ref[1, :, :d]) / dsum
    return agg + b_ref[...]


def _comb_body(acc_ref, den_ref, b_ref, w_ref, as_ref, ad_ref,
               h_ref, ss_ref, sd_ref, *, d, relu):
    agg = _combine(acc_ref, den_ref, b_ref, d)
    if relu:
        agg = jnp.maximum(agg, 0.0)
    h = _dot(agg, w_ref[...])
    h_ref[...] = h
    ss_ref[...] = _dot(h, as_ref[...])
    sd_ref[...] = _dot(h, ad_ref[...])


def _acc_spec():
    return pl.BlockSpec((NC, BR, DH), lambda i: (0, i, 0))


def _den_spec():
    return pl.BlockSpec((BR, NW), lambda i: (i, 0))


def _comb(acc, den, b, w, a_s, a_d, relu):
    d, dh = w.shape
    return pl.pallas_call(
        functools.partial(_comb_body, d=d, relu=relu),
        grid=(NB,),
        in_specs=[_acc_spec(), _den_spec(), _full_spec((1, d)),
                  _full_spec(w.shape), _full_spec((dh, 1)),
                  _full_spec((dh, 1))],
        out_specs=(_row_spec(dh), _row_spec(1), _row_spec(1)),
        out_shape=(jax.ShapeDtypeStruct((NPAD, dh), F32),
                   jax.ShapeDtypeStruct((NPAD, 1), F32),
                   jax.ShapeDtypeStruct((NPAD, 1), F32)),
    )(acc, den, b.reshape(1, -1), w, a_s.reshape(-1, 1), a_d.reshape(-1, 1))


def _t2_body(acc_ref, den_ref, b_ref, w_ref, as_ref, ad_ref, ct_ref,
             z_ref, q_ref, h_ref, ss_ref, sd_ref, *, d):
    zr = _combine(acc_ref, den_ref, b_ref, d)
    nrm = jnp.sqrt(jnp.sum(zr * zr, axis=1, keepdims=True))
    z = zr / jnp.maximum(nrm, 1e-12)
    z_ref[...] = z
    zn = jnp.sum(z * z, axis=1, keepdims=True)
    ct = ct_ref[...]
    cn = jnp.sum(ct * ct, axis=0, keepdims=True)
    dist = zn + cn - 2.0 * _dot(z, ct)
    qm = 1.0 / (1.0 + dist) + 1e-7
    q_ref[...] = qm / jnp.sum(qm, axis=1, keepdims=True)
    h = _dot(z, w_ref[...])
    h_ref[...] = h
    ss_ref[...] = _dot(h, as_ref[...])
    sd_ref[...] = _dot(h, ad_ref[...])


def _t2(acc, den, b, w, a_s, a_d, cluster_t):
    dz, k = cluster_t.shape
    dh = w.shape[1]
    return pl.pallas_call(
        functools.partial(_t2_body, d=dz),
        grid=(NB,),
        in_specs=[_acc_spec(), _den_spec(), _full_spec((1, dz)),
                  _full_spec(w.shape), _full_spec((dh, 1)),
                  _full_spec((dh, 1)), _full_spec(cluster_t.shape)],
        out_specs=(_row_spec(dz), _row_spec(k), _row_spec(dh),
                   _row_spec(1), _row_spec(1)),
        out_shape=(jax.ShapeDtypeStruct((NPAD, dz), F32),
                   jax.ShapeDtypeStruct((NPAD, k), F32),
                   jax.ShapeDtypeStruct((NPAD, dh), F32),
                   jax.ShapeDtypeStruct((NPAD, 1), F32),
                   jax.ShapeDtypeStruct((NPAD, 1), F32)),
    )(acc, den, b.reshape(1, -1), w, a_s.reshape(-1, 1), a_d.reshape(-1, 1),
      cluster_t)


def _t4_body(acc_ref, den_ref, b_ref, out_ref, *, d):
    out_ref[...] = _combine(acc_ref, den_ref, b_ref, d)


def _t4(acc, den, b, d):
    return pl.pallas_call(
        functools.partial(_t4_body, d=d),
        grid=(NB,),
        in_specs=[_acc_spec(), _den_spec(), _full_spec((1, d))],
        out_specs=_row_spec(d),
        out_shape=jax.ShapeDtypeStruct((NPAD, d), F32),
    )(acc, den, b.reshape(1, -1))


def kernel(x, edge_index, W1, as1, ad1, b1, W2, as2, ad2, b2,
           W3, as3, ad3, b3, W4, as4, ad4, b4, cluster):
    n = x.shape[0]
    d_in = x.shape[1]
    d_z = W2.shape[1]
    loops = jnp.arange(n, dtype=edge_index.dtype)
    ndummy = EPAD - (edge_index.shape[1] + n)
    src = jnp.concatenate([edge_index[0], loops, jnp.zeros((ndummy,), I32)])
    dst = jnp.concatenate([edge_index[1], loops, jnp.full((ndummy,), n, I32)])
    x_pad = jnp.zeros((NPAD, d_in), F32).at[:n].set(x)
    zrows = jnp.zeros((CHUNK, DH), F32)
    z1 = jnp.zeros((NPAD,), F32)
    W2p = jnp.zeros((d_in, DH), F32).at[:, :d_z].set(W2)
    as2p = jnp.zeros((DH,), F32).at[:d_z].set(as2)
    ad2p = jnp.zeros((DH,), F32).at[:d_z].set(ad2)

    h1, ss1, sd1 = _t0(x_pad, W1, as1, ad1)
    acc1, den1 = _edge_pass(src, dst, ss1, sd1, h1, zrows, z1)
    h2, ss2, sd2 = _comb(acc1, den1, b1, W2p, as2p, ad2p, relu=True)
    acc2, den2 = _edge_pass(src, dst, ss2, sd2, h2, zrows, z1)
    z_full, q_full, h3, ss3, sd3 = _t2(acc2, den2, b2, W3, as3, ad3, cluster.T)
    acc3, den3 = _edge_pass(src, dst, ss3, sd3, h3, zrows, z1)
    h4, ss4, sd4 = _comb(acc3, den3, b3, W4, as4, ad4, relu=True)
    acc4, den4 = _edge_pass(src, dst, ss4, sd4, h4, zrows, z1)
    x_hat = _t4(acc4, den4, b4, d_in)

    return (z_full[:n], x_hat[:n], q_full[:n])


# speedup vs baseline: 1.3757x; 1.3757x over previous
import functools

import jax
import jax.numpy as jnp
from jax import lax
from jax.experimental import pallas as pl
from jax.experimental.pallas import tpu as pltpu
from jax.experimental.pallas import tpu_sc as plsc

F32 = jnp.float32
I32 = jnp.int32
_HI = lax.Precision.HIGHEST

NC, NS, LANES = 2, 16, 16
NW = NC * NS

NNODE = 10000
DH = 128
RPT = 632
NPAD = RPT * NS
NB = 8
BR = NPAD // NB
CHUNK = 128
HALF = 64
NCHUNK = 82
KE = CHUNK * NCHUNK
EPAD = KE * NW


def _dot(a, b):
    return jnp.dot(a, b, precision=_HI, preferred_element_type=F32)


def _make_edge_kernel(d):
    nzb, nzr = divmod(RPT, HALF)
    mesh = plsc.VectorSubcoreMesh(
        core_axis_name="c", subcore_axis_name="s",
        num_cores=NC, num_subcores=NS)

    @functools.partial(
        pl.kernel,
        out_type=(jax.ShapeDtypeStruct((NC * NPAD, d), F32),
                  jax.ShapeDtypeStruct((NW * NPAD,), F32)),
        mesh=mesh,
        scratch_types=[
            pltpu.VMEM_SHARED((NPAD, d), F32),
            [pltpu.VMEM((CHUNK,), I32) for _ in range(2)],
            [pltpu.VMEM((CHUNK,), I32) for _ in range(2)],
            pltpu.VMEM((CHUNK,), F32),
            [pltpu.VMEM((HALF, d), F32) for _ in range(2)],
            [pltpu.VMEM((HALF,), I32) for _ in range(2)],
            [pltpu.VMEM((HALF,), I32) for _ in range(2)],
            pltpu.VMEM((NPAD,), F32),
            pltpu.VMEM((NPAD,), F32),
            pltpu.VMEM((NPAD,), F32),
            [pltpu.SemaphoreType.DMA for _ in range(2)],
            [pltpu.SemaphoreType.DMA for _ in range(2)],
            [pltpu.SemaphoreType.DMA for _ in range(4)],
        ],
        compiler_params=pltpu.CompilerParams(
            needs_layout_passes=False,
            use_tc_tiling_on_sc=(d == DH)),
    )
    def edge_kernel(src_hbm, dst_hbm, ss_hbm, sd_hbm, h_hbm, zrows_hbm, z1_hbm,
                    acc_hbm, den_hbm,
                    acc_sp, sxq, dxq, exv, rq, siq, diq, ssv, sdv, denv,
                    gsem, ssem, isem):
        cid = lax.axis_index("c")
        sid = lax.axis_index("s")
        wid = cid * NS + sid
        r0 = sid * RPT

        pltpu.sync_copy(ss_hbm, ssv)
        pltpu.sync_copy(sd_hbm, sdv)
        pltpu.sync_copy(z1_hbm, denv)

        pltpu.sync_copy(zrows_hbm, rq[0])
        for i in range(nzb):
            pltpu.sync_copy(rq[0], acc_sp.at[pl.ds(r0 + i * HALF, HALF)])
        pltpu.sync_copy(rq[0].at[pl.ds(0, nzr)],
                        acc_sp.at[pl.ds(r0 + nzb * HALF, nzr)])

        plsc.subcore_barrier()

        ebase = wid * KE

        def compute_ex(h, idx_s, idx_d):
            for j in range(HALF // LANES):
                si = idx_s[pl.ds(j * LANES, LANES)]
                di = idx_d[pl.ds(j * LANES, LANES)]
                t = plsc.load_gather(ssv, [si]) + plsc.load_gather(sdv, [di])
                e = jnp.maximum(t, 0.2 * t)
                ex = jnp.exp(e)
                exv[pl.ds(h * HALF + j * LANES, LANES)] = ex
                plsc.addupdate_scatter(denv, [di], ex)

        def scale_rows(h, rbuf):
            def scale_body(jj, c2):
                b = plsc.load_gather(
                    exv, [jnp.full((LANES,), h * HALF + jj, I32)])
                for dd in range(d // LANES):
                    sl = pl.ds(dd * LANES, LANES)
                    rbuf[jj, sl] = rbuf[jj, sl] * b
                return c2
            lax.fori_loop(0, HALF, scale_body, 0)

        def process_chunk(sidx, didx):
            for h in range(2):
                for k in range(HALF // LANES):
                    sl64 = pl.ds(k * LANES, LANES)
                    sl128 = pl.ds(h * HALF + k * LANES, LANES)
                    siq[h][sl64] = sidx[sl128]
                    diq[h][sl64] = didx[sl128]
            g0 = pltpu.async_copy(h_hbm.at[siq[0]], rq[0], gsem[0])
            g1 = pltpu.async_copy(h_hbm.at[siq[1]], rq[1], gsem[1])
            compute_ex(0, siq[0], diq[0])
            compute_ex(1, siq[1], diq[1])
            g0.wait()
            scale_rows(0, rq[0])
            s0 = pltpu.async_copy(rq[0], acc_sp.at[diq[0]], ssem[0], add=True)
            g1.wait()
            scale_rows(1, rq[1])
            s1 = pltpu.async_copy(rq[1], acc_sp.at[diq[1]], ssem[1], add=True)
            s0.wait()
            s1.wait()

        def pair_body(p, carry):
            bA = ebase + (2 * p) * CHUNK
            bB = bA + CHUNK
            iA0 = pltpu.async_copy(src_hbm.at[pl.ds(bA, CHUNK)], sxq[0],
                                   isem[0])
            iA1 = pltpu.async_copy(dst_hbm.at[pl.ds(bA, CHUNK)], dxq[0],
                                   isem[1])
            iB0 = pltpu.async_copy(src_hbm.at[pl.ds(bB, CHUNK)], sxq[1],
                                   isem[2])
            iB1 = pltpu.async_copy(dst_hbm.at[pl.ds(bB, CHUNK)], dxq[1],
                                   isem[3])
            iA0.wait()
            iA1.wait()
            process_chunk(sxq[0], dxq[0])
            iB0.wait()
            iB1.wait()
            process_chunk(sxq[1], dxq[1])
            return carry

        lax.fori_loop(0, NCHUNK // 2, pair_body, 0)

        pltpu.sync_copy(denv, den_hbm.at[pl.ds(wid * NPAD, NPAD)])

        plsc.subcore_barrier()

        ob = cid * NPAD + r0
        for i in range(nzb):
            b = rq[i % 2]
            pltpu.sync_copy(acc_sp.at[pl.ds(r0 + i * HALF, HALF)], b)
            pltpu.sync_copy(b, acc_hbm.at[pl.ds(ob + i * HALF, HALF)])
        pltpu.sync_copy(acc_sp.at[pl.ds(r0 + nzb * HALF, nzr)],
                        rq[1].at[pl.ds(0, nzr)])
        pltpu.sync_copy(rq[1].at[pl.ds(0, nzr)],
                        acc_hbm.at[pl.ds(ob + nzb * HALF, nzr)])

    return edge_kernel


_EK = {DH: _make_edge_kernel(DH), 32: _make_edge_kernel(32)}


def _edge_pass(src, dst, ss, sd, h, zrows, z1):
    d = h.shape[1]
    acc, den = _EK[d](src, dst, ss.reshape(-1), sd.reshape(-1), h, zrows, z1)
    return acc.reshape(NC, NPAD, d), den.reshape(NW, NPAD).T


def _t0_body(x_ref, w_ref, as_ref, ad_ref, h_ref, ss_ref, sd_ref):
    x = x_ref[...]
    nrm = jnp.sqrt(jnp.sum(x * x, axis=1, keepdims=True))
    xn = x / jnp.maximum(nrm, 1e-12)
    h = _dot(xn, w_ref[...])
    h_ref[...] = h
    ss_ref[...] = _dot(h, as_ref[...])
    sd_ref[...] = _dot(h, ad_ref[...])


def _row_spec(d):
    return pl.BlockSpec((BR, d), lambda i: (i, 0))


def _full_spec(shape):
    return pl.BlockSpec(shape, lambda i: tuple(0 for _ in shape))


def _t0(x_pad, w, a_s, a_d):
    dh = w.shape[1]
    return pl.pallas_call(
        _t0_body,
        grid=(NB,),
        in_specs=[_row_spec(x_pad.shape[1]), _full_spec(w.shape),
                  _full_spec((w.shape[1], 1)), _full_spec((w.shape[1], 1))],
        out_specs=(_row_spec(dh), _row_spec(1), _row_spec(1)),
        out_shape=(jax.ShapeDtypeStruct((NPAD, dh), F32),
                   jax.ShapeDtypeStruct((NPAD, 1), F32),
                   jax.ShapeDtypeStruct((NPAD, 1), F32)),
    )(x_pad, w, a_s.reshape(-1, 1), a_d.reshape(-1, 1))


def _acc_spec(d):
    return pl.BlockSpec((NC, BR, d), lambda i: (0, i, 0))


def _den_spec():
    return pl.BlockSpec((BR, NW), lambda i: (i, 0))


def _combine(acc_ref, den_ref, b_ref):
    dsum = jnp.sum(den_ref[...], axis=1, keepdims=True) + 1e-16
    agg = (acc_ref[0] + acc_ref[1]) / dsum
    return agg + b_ref[...]


def _comb_body(acc_ref, den_ref, b_ref, w_ref, as_ref, ad_ref,
               h_ref, ss_ref, sd_ref, *, relu):
    agg = _combine(acc_ref, den_ref, b_ref)
    if relu:
        agg = jnp.maximum(agg, 0.0)
    h = _dot(agg, w_ref[...])
    h_ref[...] = h
    ss_ref[...] = _dot(h, as_ref[...])
    sd_ref[...] = _dot(h, ad_ref[...])


def _comb(acc, den, b, w, a_s, a_d, relu):
    d, dh = w.shape
    return pl.pallas_call(
        functools.partial(_comb_body, relu=relu),
        grid=(NB,),
        in_specs=[_acc_spec(d), _den_spec(), _full_spec((1, d)),
                  _full_spec(w.shape), _full_spec((dh, 1)),
                  _full_spec((dh, 1))],
        out_specs=(_row_spec(dh), _row_spec(1), _row_spec(1)),
        out_shape=(jax.ShapeDtypeStruct((NPAD, dh), F32),
                   jax.ShapeDtypeStruct((NPAD, 1), F32),
                   jax.ShapeDtypeStruct((NPAD, 1), F32)),
    )(acc, den, b.reshape(1, -1), w, a_s.reshape(-1, 1), a_d.reshape(-1, 1))


def _t3_body(acc_ref, den_ref, b_ref, w3_ref, w4_ref, as_ref, ad_ref,
             h_ref, ss_ref, sd_ref):
    agg = _combine(acc_ref, den_ref, jnp.zeros((1, 1), F32))
    d1 = jnp.maximum(_dot(agg, w3_ref[...]) + b_ref[...], 0.0)
    h = _dot(d1, w4_ref[...])
    h_ref[...] = h
    ss_ref[...] = _dot(h, as_ref[...])
    sd_ref[...] = _dot(h, ad_ref[...])


def _t3(acc, den, b, w3, w4, a_s, a_d):
    dz, dh = w3.shape
    dh4 = w4.shape[1]
    return pl.pallas_call(
        _t3_body,
        grid=(NB,),
        in_specs=[_acc_spec(dz), _den_spec(), _full_spec((1, dh)),
                  _full_spec(w3.shape), _full_spec(w4.shape),
                  _full_spec((dh4, 1)), _full_spec((dh4, 1))],
        out_specs=(_row_spec(dh4), _row_spec(1), _row_spec(1)),
        out_shape=(jax.ShapeDtypeStruct((NPAD, dh4), F32),
                   jax.ShapeDtypeStruct((NPAD, 1), F32),
                   jax.ShapeDtypeStruct((NPAD, 1), F32)),
    )(acc, den, b.reshape(1, -1), w3, w4, a_s.reshape(-1, 1),
      a_d.reshape(-1, 1))


def _t2_body(acc_ref, den_ref, b_ref, w_ref, as_ref, ad_ref, ct_ref,
             z_ref, q_ref, ss_ref, sd_ref):
    zr = _combine(acc_ref, den_ref, b_ref)
    nrm = jnp.sqrt(jnp.sum(zr * zr, axis=1, keepdims=True))
    z = zr / jnp.maximum(nrm, 1e-12)
    z_ref[...] = z
    zn = jnp.sum(z * z, axis=1, keepdims=True)
    ct = ct_ref[...]
    cn = jnp.sum(ct * ct, axis=0, keepdims=True)
    dist = zn + cn - 2.0 * _dot(z, ct)
    qm = 1.0 / (1.0 + dist) + 1e-7
    q_ref[...] = qm / jnp.sum(qm, axis=1, keepdims=True)
    h = _dot(z, w_ref[...])
    ss_ref[...] = _dot(h, as_ref[...])
    sd_ref[...] = _dot(h, ad_ref[...])


def _t2(acc, den, b, w, a_s, a_d, cluster_t):
    dz, k = cluster_t.shape
    dh = w.shape[1]
    return pl.pallas_call(
        _t2_body,
        grid=(NB,),
        in_specs=[_acc_spec(dz), _den_spec(), _full_spec((1, dz)),
                  _full_spec(w.shape), _full_spec((dh, 1)),
                  _full_spec((dh, 1)), _full_spec(cluster_t.shape)],
        out_specs=(_row_spec(dz), _row_spec(k),
                   _row_spec(1), _row_spec(1)),
        out_shape=(jax.ShapeDtypeStruct((NPAD, dz), F32),
                   jax.ShapeDtypeStruct((NPAD, k), F32),
                   jax.ShapeDtypeStruct((NPAD, 1), F32),
                   jax.ShapeDtypeStruct((NPAD, 1), F32)),
    )(acc, den, b.reshape(1, -1), w, a_s.reshape(-1, 1), a_d.reshape(-1, 1),
      cluster_t)


def _t4_body(acc_ref, den_ref, b_ref, out_ref):
    out_ref[...] = _combine(acc_ref, den_ref, b_ref)


def _t4(acc, den, b, d):
    return pl.pallas_call(
        _t4_body,
        grid=(NB,),
        in_specs=[_acc_spec(d), _den_spec(), _full_spec((1, d))],
        out_specs=_row_spec(d),
        out_shape=jax.ShapeDtypeStruct((NPAD, d), F32),
    )(acc, den, b.reshape(1, -1))


def kernel(x, edge_index, W1, as1, ad1, b1, W2, as2, ad2, b2,
           W3, as3, ad3, b3, W4, as4, ad4, b4, cluster):
    n = x.shape[0]
    d_in = x.shape[1]
    d_z = W2.shape[1]
    loops = jnp.arange(n, dtype=edge_index.dtype)
    ndummy = EPAD - (edge_index.shape[1] + n)
    src = jnp.concatenate([edge_index[0], loops, jnp.zeros((ndummy,), I32)])
    dst = jnp.concatenate([edge_index[1], loops, jnp.full((ndummy,), n, I32)])
    x_pad = jnp.zeros((NPAD, d_in), F32).at[:n].set(x)
    zrows = jnp.zeros((HALF, DH), F32)
    zrows32 = jnp.zeros((HALF, d_z), F32)
    z1 = jnp.zeros((NPAD,), F32)

    h1, ss1, sd1 = _t0(x_pad, W1, as1, ad1)
    acc1, den1 = _edge_pass(src, dst, ss1, sd1, h1, zrows, z1)
    h2, ss2, sd2 = _comb(acc1, den1, b1, W2, as2, ad2, relu=True)
    acc2, den2 = _edge_pass(src, dst, ss2, sd2, h2, zrows32, z1)
    z_full, q_full, ss3, sd3 = _t2(acc2, den2, b2, W3, as3, ad3, cluster.T)
    acc3, den3 = _edge_pass(src, dst, ss3, sd3, z_full, zrows32, z1)
    h4, ss4, sd4 = _t3(acc3, den3, b3, W3, W4, as4, ad4)
    acc4, den4 = _edge_pass(src, dst, ss4, sd4, h4, zrows, z1)
    x_hat = _t4(acc4, den4, b4, d_in)

    return (z_full[:n], x_hat[:n], q_full[:n])


# speedup vs baseline: 1.5459x; 1.1237x over previous
import functools

import jax
import jax.numpy as jnp
from jax import lax
from jax.experimental import pallas as pl
from jax.experimental.pallas import tpu as pltpu
from jax.experimental.pallas import tpu_sc as plsc

F32 = jnp.float32
I32 = jnp.int32
_HI = lax.Precision.HIGHEST

NC, NS, LANES = 2, 16, 16
NW = NC * NS

NNODE = 10000
DH = 128
RPT = 632
NPAD = RPT * NS
NB = 8
BR = NPAD // NB
CHUNK = 128
HALF = 64
NCHUNK = 82
KE = CHUNK * NCHUNK
EPAD = KE * NW


def _dot(a, b):
    return jnp.dot(a, b, precision=_HI, preferred_element_type=F32)


def _make_edge_kernel(d):
    nzb, nzr = divmod(RPT, HALF)
    mesh = plsc.VectorSubcoreMesh(
        core_axis_name="c", subcore_axis_name="s",
        num_cores=NC, num_subcores=NS)

    @functools.partial(
        pl.kernel,
        out_type=(jax.ShapeDtypeStruct((NC * NPAD, d), F32),
                  jax.ShapeDtypeStruct((NW * NPAD,), F32)),
        mesh=mesh,
        scratch_types=[
            pltpu.VMEM_SHARED((NPAD, d), F32),
            [pltpu.VMEM((CHUNK,), I32) for _ in range(2)],
            [pltpu.VMEM((CHUNK,), I32) for _ in range(2)],
            pltpu.VMEM((CHUNK,), F32),
            [pltpu.VMEM((HALF, d), F32) for _ in range(2)],
            [pltpu.VMEM((HALF,), I32) for _ in range(2)],
            [pltpu.VMEM((HALF,), I32) for _ in range(2)],
            pltpu.VMEM((NPAD,), F32),
            pltpu.VMEM((NPAD,), F32),
            pltpu.VMEM((NPAD,), F32),
            [pltpu.SemaphoreType.DMA for _ in range(2)],
            [pltpu.SemaphoreType.DMA for _ in range(2)],
            [pltpu.SemaphoreType.DMA for _ in range(4)],
        ],
        compiler_params=pltpu.CompilerParams(
            needs_layout_passes=False,
            use_tc_tiling_on_sc=(d == DH)),
    )
    def edge_kernel(src_hbm, dst_hbm, ss_hbm, sd_hbm, h_hbm, zrows_hbm, z1_hbm,
                    acc_hbm, den_hbm,
                    acc_sp, sxq, dxq, exv, rq, siq, diq, ssv, sdv, denv,
                    gsem, ssem, isem):
        cid = lax.axis_index("c")
        sid = lax.axis_index("s")
        wid = cid * NS + sid
        r0 = sid * RPT

        pltpu.sync_copy(ss_hbm, ssv)
        pltpu.sync_copy(sd_hbm, sdv)
        pltpu.sync_copy(z1_hbm, denv)

        pltpu.sync_copy(zrows_hbm, rq[0])
        for i in range(nzb):
            pltpu.sync_copy(rq[0], acc_sp.at[pl.ds(r0 + i * HALF, HALF)])
        pltpu.sync_copy(rq[0].at[pl.ds(0, nzr)],
                        acc_sp.at[pl.ds(r0 + nzb * HALF, nzr)])

        plsc.subcore_barrier()

        ebase = wid * KE

        def compute_ex(h, idx_s, idx_d):
            for j in range(HALF // LANES):
                si = idx_s[pl.ds(j * LANES, LANES)]
                di = idx_d[pl.ds(j * LANES, LANES)]
                t = plsc.load_gather(ssv, [si]) + plsc.load_gather(sdv, [di])
                e = jnp.maximum(t, 0.2 * t)
                ex = jnp.exp(e)
                exv[pl.ds(h * HALF + j * LANES, LANES)] = ex
                plsc.addupdate_scatter(denv, [di], ex)

        def scale_rows(h, rbuf):
            @plsc.parallel_loop(0, HALF, unroll=4)
            def scale_body(jj):
                b = plsc.load_gather(
                    exv, [jnp.full((LANES,), h * HALF + jj, I32)])
                for dd in range(d // LANES):
                    sl = pl.ds(dd * LANES, LANES)
                    rbuf[jj, sl] = rbuf[jj, sl] * b

        def process_chunk(sidx, didx):
            for h in range(2):
                for k in range(HALF // LANES):
                    sl64 = pl.ds(k * LANES, LANES)
                    sl128 = pl.ds(h * HALF + k * LANES, LANES)
                    siq[h][sl64] = sidx[sl128]
                    diq[h][sl64] = didx[sl128]
            g0 = pltpu.async_copy(h_hbm.at[siq[0]], rq[0], gsem[0])
            g1 = pltpu.async_copy(h_hbm.at[siq[1]], rq[1], gsem[1])
            compute_ex(0, siq[0], diq[0])
            compute_ex(1, siq[1], diq[1])
            g0.wait()
            scale_rows(0, rq[0])
            s0 = pltpu.async_copy(rq[0], acc_sp.at[diq[0]], ssem[0], add=True)
            g1.wait()
            scale_rows(1, rq[1])
            s1 = pltpu.async_copy(rq[1], acc_sp.at[diq[1]], ssem[1], add=True)
            s0.wait()
            s1.wait()

        def pair_body(p, carry):
            bA = ebase + (2 * p) * CHUNK
            bB = bA + CHUNK
            iA0 = pltpu.async_copy(src_hbm.at[pl.ds(bA, CHUNK)], sxq[0],
                                   isem[0])
            iA1 = pltpu.async_copy(dst_hbm.at[pl.ds(bA, CHUNK)], dxq[0],
                                   isem[1])
            iB0 = pltpu.async_copy(src_hbm.at[pl.ds(bB, CHUNK)], sxq[1],
                                   isem[2])
            iB1 = pltpu.async_copy(dst_hbm.at[pl.ds(bB, CHUNK)], dxq[1],
                                   isem[3])
            iA0.wait()
            iA1.wait()
            process_chunk(sxq[0], dxq[0])
            iB0.wait()
            iB1.wait()
            process_chunk(sxq[1], dxq[1])
            return carry

        lax.fori_loop(0, NCHUNK // 2, pair_body, 0)

        pltpu.sync_copy(denv, den_hbm.at[pl.ds(wid * NPAD, NPAD)])

        plsc.subcore_barrier()

        ob = cid * NPAD + r0
        for i in range(nzb):
            b = rq[i % 2]
            pltpu.sync_copy(acc_sp.at[pl.ds(r0 + i * HALF, HALF)], b)
            pltpu.sync_copy(b, acc_hbm.at[pl.ds(ob + i * HALF, HALF)])
        pltpu.sync_copy(acc_sp.at[pl.ds(r0 + nzb * HALF, nzr)],
                        rq[1].at[pl.ds(0, nzr)])
        pltpu.sync_copy(rq[1].at[pl.ds(0, nzr)],
                        acc_hbm.at[pl.ds(ob + nzb * HALF, nzr)])

    return edge_kernel


_EK = {DH: _make_edge_kernel(DH), 32: _make_edge_kernel(32)}


def _edge_pass(src, dst, ss, sd, h, zrows, z1):
    d = h.shape[1]
    acc, den = _EK[d](src, dst, ss.reshape(-1), sd.reshape(-1), h, zrows, z1)
    return acc.reshape(NC, NPAD, d), den.reshape(NW, NPAD).T


def _t0_body(x_ref, w_ref, as_ref, ad_ref, h_ref, ss_ref, sd_ref):
    x = x_ref[...]
    nrm = jnp.sqrt(jnp.sum(x * x, axis=1, keepdims=True))
    xn = x / jnp.maximum(nrm, 1e-12)
    h = _dot(xn, w_ref[...])
    h_ref[...] = h
    ss_ref[...] = _dot(h, as_ref[...])
    sd_ref[...] = _dot(h, ad_ref[...])


def _row_spec(d):
    return pl.BlockSpec((BR, d), lambda i: (i, 0))


def _full_spec(shape):
    return pl.BlockSpec(shape, lambda i: tuple(0 for _ in shape))


def _t0(x_pad, w, a_s, a_d):
    dh = w.shape[1]
    return pl.pallas_call(
        _t0_body,
        grid=(NB,),
        in_specs=[_row_spec(x_pad.shape[1]), _full_spec(w.shape),
                  _full_spec((w.shape[1], 1)), _full_spec((w.shape[1], 1))],
        out_specs=(_row_spec(dh), _row_spec(1), _row_spec(1)),
        out_shape=(jax.ShapeDtypeStruct((NPAD, dh), F32),
                   jax.ShapeDtypeStruct((NPAD, 1), F32),
                   jax.ShapeDtypeStruct((NPAD, 1), F32)),
    )(x_pad, w, a_s.reshape(-1, 1), a_d.reshape(-1, 1))


def _acc_spec(d):
    return pl.BlockSpec((NC, BR, d), lambda i: (0, i, 0))


def _den_spec():
    return pl.BlockSpec((BR, NW), lambda i: (i, 0))


def _combine(acc_ref, den_ref, b_ref):
    dsum = jnp.sum(den_ref[...], axis=1, keepdims=True) + 1e-16
    agg = (acc_ref[0] + acc_ref[1]) / dsum
    return agg + b_ref[...]


def _comb_body(acc_ref, den_ref, b_ref, w_ref, as_ref, ad_ref,
               h_ref, ss_ref, sd_ref, *, relu):
    agg = _combine(acc_ref, den_ref, b_ref)
    if relu:
        agg = jnp.maximum(agg, 0.0)
    h = _dot(agg, w_ref[...])
    h_ref[...] = h
    ss_ref[...] = _dot(h, as_ref[...])
    sd_ref[...] = _dot(h, ad_ref[...])


def _comb(acc, den, b, w, a_s, a_d, relu):
    d, dh = w.shape
    return pl.pallas_call(
        functools.partial(_comb_body, relu=relu),
        grid=(NB,),
        in_specs=[_acc_spec(d), _den_spec(), _full_spec((1, d)),
                  _full_spec(w.shape), _full_spec((dh, 1)),
                  _full_spec((dh, 1))],
        out_specs=(_row_spec(dh), _row_spec(1), _row_spec(1)),
        out_shape=(jax.ShapeDtypeStruct((NPAD, dh), F32),
                   jax.ShapeDtypeStruct((NPAD, 1), F32),
                   jax.ShapeDtypeStruct((NPAD, 1), F32)),
    )(acc, den, b.reshape(1, -1), w, a_s.reshape(-1, 1), a_d.reshape(-1, 1))


def _t3_body(acc_ref, den_ref, b_ref, w3_ref, w4_ref, as_ref, ad_ref,
             h_ref, ss_ref, sd_ref):
    agg = _combine(acc_ref, den_ref, jnp.zeros((1, 1), F32))
    d1 = jnp.maximum(_dot(agg, w3_ref[...]) + b_ref[...], 0.0)
    h = _dot(d1, w4_ref[...])
    h_ref[...] = h
    ss_ref[...] = _dot(h, as_ref[...])
    sd_ref[...] = _dot(h, ad_ref[...])


def _t3(acc, den, b, w3, w4, a_s, a_d):
    dz, dh = w3.shape
    dh4 = w4.shape[1]
    return pl.pallas_call(
        _t3_body,
        grid=(NB,),
        in_specs=[_acc_spec(dz), _den_spec(), _full_spec((1, dh)),
                  _full_spec(w3.shape), _full_spec(w4.shape),
                  _full_spec((dh4, 1)), _full_spec((dh4, 1))],
        out_specs=(_row_spec(dh4), _row_spec(1), _row_spec(1)),
        out_shape=(jax.ShapeDtypeStruct((NPAD, dh4), F32),
                   jax.ShapeDtypeStruct((NPAD, 1), F32),
                   jax.ShapeDtypeStruct((NPAD, 1), F32)),
    )(acc, den, b.reshape(1, -1), w3, w4, a_s.reshape(-1, 1),
      a_d.reshape(-1, 1))


def _t2_body(acc_ref, den_ref, b_ref, w_ref, as_ref, ad_ref, ct_ref,
             z_ref, q_ref, ss_ref, sd_ref):
    zr = _combine(acc_ref, den_ref, b_ref)
    nrm = jnp.sqrt(jnp.sum(zr * zr, axis=1, keepdims=True))
    z = zr / jnp.maximum(nrm, 1e-12)
    z_ref[...] = z
    zn = jnp.sum(z * z, axis=1, keepdims=True)
    ct = ct_ref[...]
    cn = jnp.sum(ct * ct, axis=0, keepdims=True)
    dist = zn + cn - 2.0 * _dot(z, ct)
    qm = 1.0 / (1.0 + dist) + 1e-7
    q_ref[...] = qm / jnp.sum(qm, axis=1, keepdims=True)
    h = _dot(z, w_ref[...])
    ss_ref[...] = _dot(h, as_ref[...])
    sd_ref[...] = _dot(h, ad_ref[...])


def _t2(acc, den, b, w, a_s, a_d, cluster_t):
    dz, k = cluster_t.shape
    dh = w.shape[1]
    return pl.pallas_call(
        _t2_body,
        grid=(NB,),
        in_specs=[_acc_spec(dz), _den_spec(), _full_spec((1, dz)),
                  _full_spec(w.shape), _full_spec((dh, 1)),
                  _full_spec((dh, 1)), _full_spec(cluster_t.shape)],
        out_specs=(_row_spec(dz), _row_spec(k),
                   _row_spec(1), _row_spec(1)),
        out_shape=(jax.ShapeDtypeStruct((NPAD, dz), F32),
                   jax.ShapeDtypeStruct((NPAD, k), F32),
                   jax.ShapeDtypeStruct((NPAD, 1), F32),
                   jax.ShapeDtypeStruct((NPAD, 1), F32)),
    )(acc, den, b.reshape(1, -1), w, a_s.reshape(-1, 1), a_d.reshape(-1, 1),
      cluster_t)


def _t4_body(acc_ref, den_ref, b_ref, out_ref):
    out_ref[...] = _combine(acc_ref, den_ref, b_ref)


def _t4(acc, den, b, d):
    return pl.pallas_call(
        _t4_body,
        grid=(NB,),
        in_specs=[_acc_spec(d), _den_spec(), _full_spec((1, d))],
        out_specs=_row_spec(d),
        out_shape=jax.ShapeDtypeStruct((NPAD, d), F32),
    )(acc, den, b.reshape(1, -1))


def kernel(x, edge_index, W1, as1, ad1, b1, W2, as2, ad2, b2,
           W3, as3, ad3, b3, W4, as4, ad4, b4, cluster):
    n = x.shape[0]
    d_in = x.shape[1]
    d_z = W2.shape[1]
    loops = jnp.arange(n, dtype=edge_index.dtype)
    ndummy = EPAD - (edge_index.shape[1] + n)
    src = jnp.concatenate([edge_index[0], loops, jnp.zeros((ndummy,), I32)])
    dst = jnp.concatenate([edge_index[1], loops, jnp.full((ndummy,), n, I32)])
    x_pad = jnp.zeros((NPAD, d_in), F32).at[:n].set(x)
    zrows = jnp.zeros((HALF, DH), F32)
    zrows32 = jnp.zeros((HALF, d_z), F32)
    z1 = jnp.zeros((NPAD,), F32)

    h1, ss1, sd1 = _t0(x_pad, W1, as1, ad1)
    acc1, den1 = _edge_pass(src, dst, ss1, sd1, h1, zrows, z1)
    h2, ss2, sd2 = _comb(acc1, den1, b1, W2, as2, ad2, relu=True)
    acc2, den2 = _edge_pass(src, dst, ss2, sd2, h2, zrows32, z1)
    z_full, q_full, ss3, sd3 = _t2(acc2, den2, b2, W3, as3, ad3, cluster.T)
    acc3, den3 = _edge_pass(src, dst, ss3, sd3, z_full, zrows32, z1)
    h4, ss4, sd4 = _t3(acc3, den3, b3, W3, W4, as4, ad4)
    acc4, den4 = _edge_pass(src, dst, ss4, sd4, h4, zrows, z1)
    x_hat = _t4(acc4, den4, b4, d_in)

    return (z_full[:n], x_hat[:n], q_full[:n])
